# Initial kernel scaffold; baseline (speedup 1.0000x reference)
import jax
import jax.numpy as jnp
from jax.experimental import pallas as pl


def kernel(input_mu, input_logsig, on_states):
    raise NotImplementedError("write your pallas kernel here")


# speedup vs baseline: 1.7539x; 1.7539x over previous
import functools

import jax
import jax.numpy as jnp
from jax import lax
from jax.experimental import pallas as pl
from jax.experimental.pallas import tpu as pltpu
from jax.experimental.pallas import tpu_sc as plsc

B, D, K = 4096, 64, 8192
DT = 2 * D
BB = 256


def _dists_argmin_body(mu1_ref, logsig1_ref, t_ref, dists_ref, ind_ref, dist_ref):
    x = jnp.concatenate(
        [mu1_ref[...], jnp.exp(logsig1_ref[...])], axis=1)
    t = t_ref[...]

    dn = (((1,), (1,)), ((), ()))
    cross = lax.dot_general(x, t, dn,
                            preferred_element_type=jnp.float32,
                            precision=lax.Precision.HIGHEST)
    n1 = jnp.sum(x * x, axis=1, keepdims=True)
    n2 = jnp.sum(t * t, axis=1)
    d = n1 + n2[None, :] - 2.0 * cross
    dists_ref[...] = d

    row_min = jnp.min(d, axis=1, keepdims=True)
    col = lax.broadcasted_iota(jnp.int32, d.shape, 1)
    row_arg = jnp.min(jnp.where(d == row_min, col, K), axis=1)
    ind_ref[...] = row_arg
    dist_ref[...] = row_min


def _dists_argmin(input_mu, input_logsig, table):
    return pl.pallas_call(
        _dists_argmin_body,
        grid=(B // BB,),
        in_specs=[
            pl.BlockSpec((BB, D), lambda i: (i, 0)),
            pl.BlockSpec((BB, D), lambda i: (i, 0)),
            pl.BlockSpec((K, DT), lambda i: (0, 0)),
        ],
        out_specs=[
            pl.BlockSpec((BB, K), lambda i: (i, 0)),
            pl.BlockSpec((BB,), lambda i: (i,)),
            pl.BlockSpec((BB, 1), lambda i: (i, 0)),
        ],
        out_shape=[
            jax.ShapeDtypeStruct((B, K), jnp.float32),
            jax.ShapeDtypeStruct((B,), jnp.int32),
            jax.ShapeDtypeStruct((B, 1), jnp.float32),
        ],
    )(input_mu, input_logsig, table)


def _make_sc_gather():
    info = plsc.get_sparse_core_info()
    nc, ns = info.num_cores, info.num_subcores
    nw = nc * ns
    bpw = B // nw
    mesh = plsc.VectorSubcoreMesh(core_axis_name="c", subcore_axis_name="s")

    @functools.partial(
        pl.kernel,
        mesh=mesh,
        out_type=jax.ShapeDtypeStruct((B, DT), jnp.float32),
        scratch_types=[
            pltpu.VMEM((bpw,), jnp.int32),
            pltpu.VMEM((bpw, DT), jnp.float32),
            pltpu.SemaphoreType.DMA,
        ],
    )
    def gather(table_hbm, idx_hbm, out_hbm, idx_v, rows_v, sem):
        wid = lax.axis_index("s") * nc + lax.axis_index("c")
        base = wid * bpw
        pltpu.sync_copy(idx_hbm.at[pl.ds(base, bpw)], idx_v)
        pltpu.async_copy(table_hbm.at[idx_v], rows_v, sem).wait()
        pltpu.sync_copy(rows_v, out_hbm.at[pl.ds(base, bpw)])

    return gather


_sc_gather = _make_sc_gather()


@jax.jit
def kernel(input_mu, input_logsig, on_states):
    table = on_states.transpose(0, 2, 1).reshape(K, DT)
    dists, ind, dist = _dists_argmin(input_mu, input_logsig, table)
    q = _sc_gather(table, ind)
    return ((q[:, :D], q[:, D:]), dists, dist)
